# Initial kernel scaffold; baseline (speedup 1.0000x reference)
import jax
import jax.numpy as jnp
from jax.experimental import pallas as pl


def kernel(x, edge_index, W1, b1, W2, b2):
    raise NotImplementedError("write your pallas kernel here")


# speedup vs baseline: 2.3899x; 2.3899x over previous
import functools

import jax
import jax.numpy as jnp
from jax import lax
from jax.experimental import pallas as pl
from jax.experimental.pallas import tpu as pltpu
from jax.experimental.pallas import tpu_sc as plsc

N = 10000
D = 256
E = 160000
NW = 32
EPW = E // NW
B = 2000
RB = 1000

_mesh = lambda: plsc.VectorSubcoreMesh(
    core_axis_name="c", subcore_axis_name="s", num_cores=2, num_subcores=16)
_sc_params = pltpu.CompilerParams(
    needs_layout_passes=False, use_tc_tiling_on_sc=False)


@functools.partial(
    pl.kernel,
    out_type=(
        jax.ShapeDtypeStruct((NW, N), jnp.float32),
        jax.ShapeDtypeStruct((E,), jnp.int32),
        jax.ShapeDtypeStruct((E,), jnp.int32),
    ),
    mesh=_mesh(),
    scratch_types=[
        pltpu.VMEM((EPW + 16,), jnp.int32),
        pltpu.VMEM((EPW + 16,), jnp.int32),
        pltpu.VMEM((EPW + 16,), jnp.int32),
        pltpu.VMEM((8 * N,), jnp.float32),
    ],
    compiler_params=_sc_params,
)
def _deg_kernel(src_hbm, dst_hbm, degp_hbm, src32_hbm, dst8_hbm,
                srcb, dstb, d8b, acc):
    wid = lax.axis_index("s") * 2 + lax.axis_index("c")
    base = wid * EPW
    pltpu.sync_copy(src_hbm.at[pl.ds(base, EPW)], srcb.at[pl.ds(0, EPW)])
    pltpu.sync_copy(dst_hbm.at[pl.ds(base, EPW)], dstb.at[pl.ds(0, EPW)])

    lanes = lax.iota(jnp.int32, 16)
    offs = (lanes & 7) * N
    mlo = lanes < 8
    mhi = lanes >= 8
    ones = jnp.ones((16,), jnp.float32)
    zero = jnp.zeros((16,), jnp.float32)

    def zb(i, _):
        for u in range(8):
            acc[pl.ds((i * 8 + u) * 16, 16)] = zero
        return 0
    lax.fori_loop(0, (8 * N) // 128, zb, 0)

    nfull = EPW // 16

    def eb(i, _):
        for u in range(8):
            j = (i * 8 + u) * 16
            sv = srcb[pl.ds(j, 16)]
            srcb[pl.ds(j, 16)] = sv * 32
            dv = dstb[pl.ds(j, 16)]
            d8b[pl.ds(j, 16)] = dv * 8
            addr = dv + offs
            plsc.addupdate_scatter(acc, [addr], ones, mask=mlo)
            plsc.addupdate_scatter(acc, [addr], ones, mask=mhi)
        return 0
    lax.fori_loop(0, nfull // 8, eb, 0)

    j = nfull * 16
    sv = srcb[pl.ds(j, 16)]
    srcb[pl.ds(j, 16)] = sv * 32
    dv = dstb[pl.ds(j, 16)]
    d8b[pl.ds(j, 16)] = dv * 8
    plsc.addupdate_scatter(acc, [dv + offs], ones, mask=mlo)

    def rb(i, _):
        s = acc[pl.ds(i * 16, 16)]
        for k in range(1, 8):
            s = s + acc[pl.ds(k * N + i * 16, 16)]
        acc[pl.ds(i * 16, 16)] = s
        return 0
    lax.fori_loop(0, N // 16, rb, 0)

    pltpu.sync_copy(srcb.at[pl.ds(0, EPW)], src32_hbm.at[pl.ds(base, EPW)])
    pltpu.sync_copy(d8b.at[pl.ds(0, EPW)], dst8_hbm.at[pl.ds(base, EPW)])
    pltpu.sync_copy(acc.at[pl.ds(0, N)], degp_hbm.at[wid])


def _make_agg(K):
    npass = K // NW

    @functools.partial(
        pl.kernel,
        out_type=jax.ShapeDtypeStruct((K, 8 * N), jnp.float32),
        mesh=_mesh(),
        scratch_types=[
            pltpu.VMEM((B,), jnp.int32),
            pltpu.VMEM((B,), jnp.int32),
            pltpu.VMEM((B, 8), jnp.float32),
            pltpu.VMEM((8 * N,), jnp.float32),
            pltpu.SemaphoreType.DMA,
        ],
        compiler_params=_sc_params,
    )
    def agg(xflat_hbm, srck_hbm, dst8_hbm, out_hbm, idxb, dstb, rows, acc, sem):
        wid = lax.axis_index("s") * 2 + lax.axis_index("c")
        lanes = lax.iota(jnp.int32, 16)
        feat = lanes & 7
        half = lanes >> 3
        mlo = lanes < 8
        mhi = lanes >= 8
        zero = jnp.zeros((16,), jnp.float32)

        for p in range(npass):
            chunk = wid + NW * p

            def zb(i, _):
                for u in range(8):
                    acc[pl.ds((i * 8 + u) * 16, 16)] = zero
                return 0
            lax.fori_loop(0, (8 * N) // 128, zb, 0)

            def bb(b, _):
                off = b * B
                pltpu.sync_copy(srck_hbm.at[pl.ds(off, B)], idxb)
                pltpu.sync_copy(dst8_hbm.at[pl.ds(off, B)], dstb)

                def ib(i, _):
                    for u in range(5):
                        j = (i * 5 + u) * 16
                        idxb[pl.ds(j, 16)] = idxb[pl.ds(j, 16)] + chunk
                    return 0
                lax.fori_loop(0, B // 80, ib, 0)

                pltpu.async_copy(xflat_hbm.at[idxb], rows, sem).wait()

                def eb(i, _):
                    for u in range(8):
                        k = i * 8 + u
                        ev = half + 2 * k
                        dstv = plsc.load_gather(dstb, [ev])
                        addr = dstv + feat
                        row = plsc.load_gather(rows, [ev, feat])
                        plsc.addupdate_scatter(acc, [addr], row, mask=mlo)
                        plsc.addupdate_scatter(acc, [addr], row, mask=mhi)
                    return 0
                lax.fori_loop(0, B // 16, eb, 0)
                return 0
            lax.fori_loop(0, E // B, bb, 0)

            pltpu.sync_copy(acc, out_hbm.at[chunk])
    return agg


_agg32 = _make_agg(32)
_agg64 = _make_agg(64)


def _tc0_body(degp_ref, dinv_ref):
    deg = jnp.sum(degp_ref[...], axis=0) + 1.0
    dinv_ref[...] = lax.rsqrt(deg)[:, None]


def _tc1_body(dinv_ref, x_ref, w1_ref, xws_ref):
    xw = jnp.dot(x_ref[...], w1_ref[...], preferred_element_type=jnp.float32)
    xws_ref[...] = xw * dinv_ref[...]


def _tc2_body(agg_ref, xws_ref, dinv_ref, b1_ref, w2_ref, xws2_ref):
    h = jnp.maximum(dinv_ref[...] * (agg_ref[...] + xws_ref[...]) + b1_ref[...], 0.0)
    xw2 = jnp.dot(h, w2_ref[...], preferred_element_type=jnp.float32)
    xws2_ref[...] = xw2 * dinv_ref[...]


def _tc3_body(agg2_ref, xws2_ref, dinv_ref, b2_ref, out_ref):
    i = pl.program_id(0)
    h2 = jnp.maximum(dinv_ref[...] * (agg2_ref[...] + xws2_ref[...]) + b2_ref[...], 0.0)
    part = jnp.sum(h2, axis=0, keepdims=True)

    @pl.when(i == 0)
    def _():
        out_ref[...] = part

    @pl.when(i > 0)
    def _():
        out_ref[...] = out_ref[...] + part

    @pl.when(i == N // RB - 1)
    def _():
        out_ref[...] = out_ref[...] * (1.0 / N)


_tc0 = pl.pallas_call(
    _tc0_body,
    in_specs=[pl.BlockSpec((NW, N), lambda: (0, 0))],
    out_specs=pl.BlockSpec((N, 1), lambda: (0, 0)),
    out_shape=jax.ShapeDtypeStruct((N, 1), jnp.float32),
)

_tc1 = pl.pallas_call(
    _tc1_body,
    grid=(N // RB,),
    in_specs=[
        pl.BlockSpec((RB, 1), lambda i: (i, 0)),
        pl.BlockSpec((RB, D), lambda i: (i, 0)),
        pl.BlockSpec((D, D), lambda i: (0, 0)),
    ],
    out_specs=pl.BlockSpec((RB, D), lambda i: (i, 0)),
    out_shape=jax.ShapeDtypeStruct((N, D), jnp.float32),
)

_tc2 = pl.pallas_call(
    _tc2_body,
    grid=(N // RB,),
    in_specs=[
        pl.BlockSpec((RB, D), lambda i: (i, 0)),
        pl.BlockSpec((RB, D), lambda i: (i, 0)),
        pl.BlockSpec((RB, 1), lambda i: (i, 0)),
        pl.BlockSpec((1, D), lambda i: (0, 0)),
        pl.BlockSpec((D, 2 * D), lambda i: (0, 0)),
    ],
    out_specs=pl.BlockSpec((RB, 2 * D), lambda i: (i, 0)),
    out_shape=jax.ShapeDtypeStruct((N, 2 * D), jnp.float32),
)

_tc3 = pl.pallas_call(
    _tc3_body,
    grid=(N // RB,),
    in_specs=[
        pl.BlockSpec((RB, 2 * D), lambda i: (i, 0)),
        pl.BlockSpec((RB, 2 * D), lambda i: (i, 0)),
        pl.BlockSpec((RB, 1), lambda i: (i, 0)),
        pl.BlockSpec((1, 2 * D), lambda i: (0, 0)),
    ],
    out_specs=pl.BlockSpec((1, 2 * D), lambda i: (0, 0)),
    out_shape=jax.ShapeDtypeStruct((1, 2 * D), jnp.float32),
)


def kernel(x, edge_index, W1, b1, W2, b2):
    src = edge_index[0].astype(jnp.int32)
    dst = edge_index[1].astype(jnp.int32)
    degp, src32, dst8 = _deg_kernel(src, dst)

    dinv = _tc0(degp)
    xws1 = _tc1(dinv, x, W1)
    agg1 = _agg32(xws1.reshape(N * 32, 8), src32, dst8)
    agg1t = agg1.reshape(32, N, 8).transpose(1, 0, 2).reshape(N, D)

    xws2 = _tc2(agg1t, xws1, dinv, b1.reshape(1, D), W2)
    agg2 = _agg64(xws2.reshape(N * 64, 8), src32 + src32, dst8)
    agg2t = agg2.reshape(64, N, 8).transpose(1, 0, 2).reshape(N, 2 * D)

    out = _tc3(agg2t, xws2, dinv, b2.reshape(1, 2 * D))
    return out.reshape(2 * D)


# speedup vs baseline: 3.9752x; 1.6633x over previous
import functools

import jax
import jax.numpy as jnp
from jax import lax
from jax.experimental import pallas as pl
from jax.experimental.pallas import tpu as pltpu
from jax.experimental.pallas import tpu_sc as plsc

N = 10000
D = 256
E = 160000
NW = 32
EPW = E // NW
B = 2000
RB = 1000

_mesh = lambda: plsc.VectorSubcoreMesh(
    core_axis_name="c", subcore_axis_name="s", num_cores=2, num_subcores=16)
_sc_params = pltpu.CompilerParams(
    needs_layout_passes=False, use_tc_tiling_on_sc=False)


@functools.partial(
    pl.kernel,
    out_type=(
        jax.ShapeDtypeStruct((NW, N), jnp.float32),
        jax.ShapeDtypeStruct((E,), jnp.int32),
        jax.ShapeDtypeStruct((E,), jnp.int32),
    ),
    mesh=_mesh(),
    scratch_types=[
        pltpu.VMEM((EPW + 16,), jnp.int32),
        pltpu.VMEM((EPW + 16,), jnp.int32),
        pltpu.VMEM((EPW + 16,), jnp.int32),
        pltpu.VMEM((8 * N,), jnp.float32),
    ],
    compiler_params=_sc_params,
)
def _deg_kernel(src_hbm, dst_hbm, degp_hbm, src32_hbm, dst8_hbm,
                srcb, dstb, d8b, acc):
    wid = lax.axis_index("s") * 2 + lax.axis_index("c")
    base = wid * EPW
    pltpu.sync_copy(src_hbm.at[pl.ds(base, EPW)], srcb.at[pl.ds(0, EPW)])
    pltpu.sync_copy(dst_hbm.at[pl.ds(base, EPW)], dstb.at[pl.ds(0, EPW)])

    lanes = lax.iota(jnp.int32, 16)
    offs = (lanes & 7) * N
    mlo = lanes < 8
    mhi = lanes >= 8
    ones = jnp.ones((16,), jnp.float32)
    zero = jnp.zeros((16,), jnp.float32)

    @plsc.parallel_loop(0, (8 * N) // 16, unroll=8)
    def _(i):
        acc[pl.ds(i * 16, 16)] = zero

    nfull = EPW // 16

    @plsc.parallel_loop(0, nfull, unroll=8)
    def _(i):
        j = i * 16
        sv = srcb[pl.ds(j, 16)]
        srcb[pl.ds(j, 16)] = sv * 32
        dv = dstb[pl.ds(j, 16)]
        d8b[pl.ds(j, 16)] = dv * 8
        addr = dv + offs
        plsc.addupdate_scatter(acc, [addr], ones, mask=mlo)
        plsc.addupdate_scatter(acc, [addr], ones, mask=mhi)

    j = nfull * 16
    sv = srcb[pl.ds(j, 16)]
    srcb[pl.ds(j, 16)] = sv * 32
    dv = dstb[pl.ds(j, 16)]
    d8b[pl.ds(j, 16)] = dv * 8
    plsc.addupdate_scatter(acc, [dv + offs], ones, mask=mlo)

    @plsc.parallel_loop(0, N // 16, unroll=4)
    def _(i):
        s = acc[pl.ds(i * 16, 16)]
        for k in range(1, 8):
            s = s + acc[pl.ds(k * N + i * 16, 16)]
        acc[pl.ds(i * 16, 16)] = s

    pltpu.sync_copy(srcb.at[pl.ds(0, EPW)], src32_hbm.at[pl.ds(base, EPW)])
    pltpu.sync_copy(d8b.at[pl.ds(0, EPW)], dst8_hbm.at[pl.ds(base, EPW)])
    pltpu.sync_copy(acc.at[pl.ds(0, N)], degp_hbm.at[wid])


def _make_agg(K):
    npass = K // NW

    @functools.partial(
        pl.kernel,
        out_type=jax.ShapeDtypeStruct((K, 8 * N), jnp.float32),
        mesh=_mesh(),
        scratch_types=[
            pltpu.VMEM((B,), jnp.int32),
            pltpu.VMEM((B,), jnp.int32),
            pltpu.VMEM((B, 8), jnp.float32),
            pltpu.VMEM((8 * N,), jnp.float32),
            pltpu.SemaphoreType.DMA,
        ],
        compiler_params=_sc_params,
    )
    def agg(xflat_hbm, srck_hbm, dst8_hbm, out_hbm, idxb, dstb, rows, acc, sem):
        wid = lax.axis_index("s") * 2 + lax.axis_index("c")
        lanes = lax.iota(jnp.int32, 16)
        feat = lanes & 7
        half = lanes >> 3
        mlo = lanes < 8
        mhi = lanes >= 8
        zero = jnp.zeros((16,), jnp.float32)

        nview = N * K - K + 1
        for p in range(npass):
            chunk = wid + NW * p
            xview = xflat_hbm.at[pl.ds(chunk, nview)]

            @plsc.parallel_loop(0, (8 * N) // 16, unroll=8)
            def _(i):
                acc[pl.ds(i * 16, 16)] = zero

            def bb(b, _):
                off = b * B
                pltpu.sync_copy(srck_hbm.at[pl.ds(off, B)], idxb)
                pltpu.sync_copy(dst8_hbm.at[pl.ds(off, B)], dstb)
                pltpu.async_copy(xview.at[idxb], rows, sem).wait()

                @plsc.parallel_loop(0, B // 2, unroll=8)
                def _(k):
                    ev = half + 2 * k
                    dstv = plsc.load_gather(dstb, [ev])
                    addr = dstv + feat
                    row = plsc.load_gather(rows, [ev, feat])
                    plsc.addupdate_scatter(acc, [addr], row, mask=mlo)
                    plsc.addupdate_scatter(acc, [addr], row, mask=mhi)
                return 0
            lax.fori_loop(0, E // B, bb, 0)

            pltpu.sync_copy(acc, out_hbm.at[chunk])
    return agg


_agg32 = _make_agg(32)
_agg64 = _make_agg(64)


def _tc0_body(degp_ref, dinv_ref):
    deg = jnp.sum(degp_ref[...], axis=0) + 1.0
    dinv_ref[...] = lax.rsqrt(deg)[:, None]


def _tc1_body(dinv_ref, x_ref, w1_ref, xws_ref):
    xw = jnp.dot(x_ref[...], w1_ref[...], preferred_element_type=jnp.float32)
    xws_ref[...] = xw * dinv_ref[...]


def _tc2_body(agg_ref, xws_ref, dinv_ref, b1_ref, w2_ref, xws2_ref):
    h = jnp.maximum(dinv_ref[...] * (agg_ref[...] + xws_ref[...]) + b1_ref[...], 0.0)
    xw2 = jnp.dot(h, w2_ref[...], preferred_element_type=jnp.float32)
    xws2_ref[...] = xw2 * dinv_ref[...]


def _tc3_body(agg2_ref, xws2_ref, dinv_ref, b2_ref, out_ref):
    i = pl.program_id(0)
    h2 = jnp.maximum(dinv_ref[...] * (agg2_ref[...] + xws2_ref[...]) + b2_ref[...], 0.0)
    part = jnp.sum(h2, axis=0, keepdims=True)

    @pl.when(i == 0)
    def _():
        out_ref[...] = part

    @pl.when(i > 0)
    def _():
        out_ref[...] = out_ref[...] + part

    @pl.when(i == N // RB - 1)
    def _():
        out_ref[...] = out_ref[...] * (1.0 / N)


_tc0 = pl.pallas_call(
    _tc0_body,
    in_specs=[pl.BlockSpec((NW, N), lambda: (0, 0))],
    out_specs=pl.BlockSpec((N, 1), lambda: (0, 0)),
    out_shape=jax.ShapeDtypeStruct((N, 1), jnp.float32),
)

_tc1 = pl.pallas_call(
    _tc1_body,
    grid=(N // RB,),
    in_specs=[
        pl.BlockSpec((RB, 1), lambda i: (i, 0)),
        pl.BlockSpec((RB, D), lambda i: (i, 0)),
        pl.BlockSpec((D, D), lambda i: (0, 0)),
    ],
    out_specs=pl.BlockSpec((RB, D), lambda i: (i, 0)),
    out_shape=jax.ShapeDtypeStruct((N, D), jnp.float32),
)

_tc2 = pl.pallas_call(
    _tc2_body,
    grid=(N // RB,),
    in_specs=[
        pl.BlockSpec((RB, D), lambda i: (i, 0)),
        pl.BlockSpec((RB, D), lambda i: (i, 0)),
        pl.BlockSpec((RB, 1), lambda i: (i, 0)),
        pl.BlockSpec((1, D), lambda i: (0, 0)),
        pl.BlockSpec((D, 2 * D), lambda i: (0, 0)),
    ],
    out_specs=pl.BlockSpec((RB, 2 * D), lambda i: (i, 0)),
    out_shape=jax.ShapeDtypeStruct((N, 2 * D), jnp.float32),
)

_tc3 = pl.pallas_call(
    _tc3_body,
    grid=(N // RB,),
    in_specs=[
        pl.BlockSpec((RB, 2 * D), lambda i: (i, 0)),
        pl.BlockSpec((RB, 2 * D), lambda i: (i, 0)),
        pl.BlockSpec((RB, 1), lambda i: (i, 0)),
        pl.BlockSpec((1, 2 * D), lambda i: (0, 0)),
    ],
    out_specs=pl.BlockSpec((1, 2 * D), lambda i: (0, 0)),
    out_shape=jax.ShapeDtypeStruct((1, 2 * D), jnp.float32),
)


def kernel(x, edge_index, W1, b1, W2, b2):
    src = edge_index[0].astype(jnp.int32)
    dst = edge_index[1].astype(jnp.int32)
    degp, src32, dst8 = _deg_kernel(src, dst)

    dinv = _tc0(degp)
    xws1 = _tc1(dinv, x, W1)
    agg1 = _agg32(xws1.reshape(N * 32, 8), src32, dst8)
    agg1t = agg1.reshape(32, N, 8).transpose(1, 0, 2).reshape(N, D)

    xws2 = _tc2(agg1t, xws1, dinv, b1.reshape(1, D), W2)
    agg2 = _agg64(xws2.reshape(N * 64, 8), src32 + src32, dst8)
    agg2t = agg2.reshape(64, N, 8).transpose(1, 0, 2).reshape(N, 2 * D)

    out = _tc3(agg2t, xws2, dinv, b2.reshape(1, 2 * D))
    return out.reshape(2 * D)


# speedup vs baseline: 7.6867x; 1.9337x over previous
import functools

import jax
import jax.numpy as jnp
from jax import lax
from jax.experimental import pallas as pl
from jax.experimental.pallas import tpu as pltpu
from jax.experimental.pallas import tpu_sc as plsc

N = 10000
D = 256
E = 160000
NW = 32
EPW = E // NW
B = 2000
RB = 1000

_mesh = lambda: plsc.VectorSubcoreMesh(
    core_axis_name="c", subcore_axis_name="s", num_cores=2, num_subcores=16)
_sc_params = pltpu.CompilerParams(
    needs_layout_passes=False, use_tc_tiling_on_sc=False)


@functools.partial(
    pl.kernel,
    out_type=(
        jax.ShapeDtypeStruct((NW, N), jnp.float32),
        jax.ShapeDtypeStruct((E,), jnp.int32),
        jax.ShapeDtypeStruct((E,), jnp.int32),
    ),
    mesh=_mesh(),
    scratch_types=[
        pltpu.VMEM((EPW + 16,), jnp.int32),
        pltpu.VMEM((EPW + 16,), jnp.int32),
        pltpu.VMEM((EPW + 16,), jnp.int32),
        pltpu.VMEM((8 * N,), jnp.float32),
    ],
    compiler_params=_sc_params,
)
def _deg_kernel(src_hbm, dst_hbm, degp_hbm, src32_hbm, dst8_hbm,
                srcb, dstb, d8b, acc):
    wid = lax.axis_index("s") * 2 + lax.axis_index("c")
    base = wid * EPW
    pltpu.sync_copy(src_hbm.at[pl.ds(base, EPW)], srcb.at[pl.ds(0, EPW)])
    pltpu.sync_copy(dst_hbm.at[pl.ds(base, EPW)], dstb.at[pl.ds(0, EPW)])

    lanes = lax.iota(jnp.int32, 16)
    offs = (lanes & 7) * N
    mlo = lanes < 8
    mhi = lanes >= 8
    ones = jnp.ones((16,), jnp.float32)
    zero = jnp.zeros((16,), jnp.float32)

    @plsc.parallel_loop(0, (8 * N) // 16, unroll=8)
    def _(i):
        acc[pl.ds(i * 16, 16)] = zero

    nfull = EPW // 16

    @plsc.parallel_loop(0, nfull, unroll=8)
    def _(i):
        j = i * 16
        sv = srcb[pl.ds(j, 16)]
        srcb[pl.ds(j, 16)] = sv * 32
        dv = dstb[pl.ds(j, 16)]
        d8b[pl.ds(j, 16)] = dv * 8
        addr = dv + offs
        plsc.addupdate_scatter(acc, [addr], ones, mask=mlo)
        plsc.addupdate_scatter(acc, [addr], ones, mask=mhi)

    j = nfull * 16
    sv = srcb[pl.ds(j, 16)]
    srcb[pl.ds(j, 16)] = sv * 32
    dv = dstb[pl.ds(j, 16)]
    d8b[pl.ds(j, 16)] = dv * 8
    plsc.addupdate_scatter(acc, [dv + offs], ones, mask=mlo)

    @plsc.parallel_loop(0, N // 16, unroll=4)
    def _(i):
        s = acc[pl.ds(i * 16, 16)]
        for k in range(1, 8):
            s = s + acc[pl.ds(k * N + i * 16, 16)]
        acc[pl.ds(i * 16, 16)] = s

    pltpu.sync_copy(srcb.at[pl.ds(0, EPW)], src32_hbm.at[pl.ds(base, EPW)])
    pltpu.sync_copy(d8b.at[pl.ds(0, EPW)], dst8_hbm.at[pl.ds(base, EPW)])
    pltpu.sync_copy(acc.at[pl.ds(0, N)], degp_hbm.at[wid])


def _make_agg(K):
    npass = K // NW

    nb = E // B
    nb2 = nb // 2

    @functools.partial(
        pl.kernel,
        out_type=jax.ShapeDtypeStruct((K, 8 * N), jnp.float32),
        mesh=_mesh(),
        scratch_types=[
            pltpu.VMEM((2, B), jnp.int32),
            pltpu.VMEM((2, B), jnp.int32),
            pltpu.VMEM((2, B, 8), jnp.float32),
            pltpu.VMEM((8 * N,), jnp.float32),
            pltpu.SemaphoreType.DMA,
            pltpu.SemaphoreType.DMA,
            pltpu.SemaphoreType.DMA,
            pltpu.SemaphoreType.DMA,
            pltpu.SemaphoreType.DMA,
            pltpu.SemaphoreType.DMA,
        ],
        compiler_params=_sc_params,
    )
    def agg(xflat_hbm, srck_hbm, dst8_hbm, out_hbm, idxb, dstb, rows, acc,
            s0, s1, g0, g1, d0, d1):
        wid = lax.axis_index("s") * 2 + lax.axis_index("c")
        lanes = lax.iota(jnp.int32, 16)
        feat = lanes & 7
        half = lanes >> 3
        mlo = lanes < 8
        mhi = lanes >= 8
        zero = jnp.zeros((16,), jnp.float32)

        def process(dref, rref):
            @plsc.parallel_loop(0, B // 2, unroll=8)
            def _(k):
                ev = half + 2 * k
                dstv = plsc.load_gather(dref, [ev])
                addr = dstv + feat
                row = plsc.load_gather(rref, [ev, feat])
                plsc.addupdate_scatter(acc, [addr], row, mask=mlo)
                plsc.addupdate_scatter(acc, [addr], row, mask=mhi)

        nview = N * K - K + 1
        for p in range(npass):
            chunk = wid + NW * p
            xview = xflat_hbm.at[pl.ds(chunk, nview)]

            @plsc.parallel_loop(0, (8 * N) // 16, unroll=8)
            def _(i):
                acc[pl.ds(i * 16, 16)] = zero

            pltpu.sync_copy(srck_hbm.at[pl.ds(0, B)], idxb.at[0])
            pltpu.async_copy(xview.at[idxb.at[0]], rows.at[0], g0)
            pltpu.async_copy(srck_hbm.at[pl.ds(B, B)], idxb.at[1], s1)
            pltpu.async_copy(dst8_hbm.at[pl.ds(0, B)], dstb.at[0], d0)
            pltpu.async_copy(dst8_hbm.at[pl.ds(B, B)], dstb.at[1], d1)

            def bb(k, _):
                off = 2 * k * B
                more = k < nb2 - 1
                pltpu.make_async_copy(
                    srck_hbm.at[pl.ds(off + B, B)], idxb.at[1], s1).wait()
                pltpu.async_copy(xview.at[idxb.at[1]], rows.at[1], g1)
                pltpu.make_async_copy(
                    xview.at[idxb.at[0]], rows.at[0], g0).wait()

                @pl.when(more)
                def _():
                    pltpu.async_copy(
                        srck_hbm.at[pl.ds(off + 2 * B, B)], idxb.at[0], s0)
                pltpu.make_async_copy(
                    dst8_hbm.at[pl.ds(off, B)], dstb.at[0], d0).wait()
                process(dstb.at[0], rows.at[0])

                @pl.when(more)
                def _():
                    pltpu.async_copy(
                        dst8_hbm.at[pl.ds(off + 2 * B, B)], dstb.at[0], d0)
                    pltpu.make_async_copy(
                        srck_hbm.at[pl.ds(off + 2 * B, B)], idxb.at[0], s0).wait()
                    pltpu.async_copy(xview.at[idxb.at[0]], rows.at[0], g0)
                pltpu.make_async_copy(
                    xview.at[idxb.at[1]], rows.at[1], g1).wait()

                @pl.when(more)
                def _():
                    pltpu.async_copy(
                        srck_hbm.at[pl.ds(off + 3 * B, B)], idxb.at[1], s1)
                pltpu.make_async_copy(
                    dst8_hbm.at[pl.ds(off + B, B)], dstb.at[1], d1).wait()
                process(dstb.at[1], rows.at[1])

                @pl.when(more)
                def _():
                    pltpu.async_copy(
                        dst8_hbm.at[pl.ds(off + 3 * B, B)], dstb.at[1], d1)
                return 0
            lax.fori_loop(0, nb2, bb, 0)

            pltpu.sync_copy(acc, out_hbm.at[chunk])
    return agg


_agg32 = _make_agg(32)
_agg64 = _make_agg(64)


def _tc0_body(degp_ref, dinv_ref):
    deg = jnp.sum(degp_ref[...], axis=0) + 1.0
    dinv_ref[...] = lax.rsqrt(deg)[:, None]


def _tc1_body(dinv_ref, x_ref, w1_ref, xws_ref):
    xw = jnp.dot(x_ref[...], w1_ref[...], preferred_element_type=jnp.float32)
    xws_ref[...] = xw * dinv_ref[...]


def _tc2_body(agg_ref, xws_ref, dinv_ref, b1_ref, w2_ref, xws2_ref):
    h = jnp.maximum(dinv_ref[...] * (agg_ref[...] + xws_ref[...]) + b1_ref[...], 0.0)
    xw2 = jnp.dot(h, w2_ref[...], preferred_element_type=jnp.float32)
    xws2_ref[...] = xw2 * dinv_ref[...]


def _tc3_body(agg2_ref, xws2_ref, dinv_ref, b2_ref, out_ref):
    i = pl.program_id(0)
    h2 = jnp.maximum(dinv_ref[...] * (agg2_ref[...] + xws2_ref[...]) + b2_ref[...], 0.0)
    part = jnp.sum(h2, axis=0, keepdims=True)

    @pl.when(i == 0)
    def _():
        out_ref[...] = part

    @pl.when(i > 0)
    def _():
        out_ref[...] = out_ref[...] + part

    @pl.when(i == N // RB - 1)
    def _():
        out_ref[...] = out_ref[...] * (1.0 / N)


_tc0 = pl.pallas_call(
    _tc0_body,
    in_specs=[pl.BlockSpec((NW, N), lambda: (0, 0))],
    out_specs=pl.BlockSpec((N, 1), lambda: (0, 0)),
    out_shape=jax.ShapeDtypeStruct((N, 1), jnp.float32),
)

_tc1 = pl.pallas_call(
    _tc1_body,
    grid=(N // RB,),
    in_specs=[
        pl.BlockSpec((RB, 1), lambda i: (i, 0)),
        pl.BlockSpec((RB, D), lambda i: (i, 0)),
        pl.BlockSpec((D, D), lambda i: (0, 0)),
    ],
    out_specs=pl.BlockSpec((RB, D), lambda i: (i, 0)),
    out_shape=jax.ShapeDtypeStruct((N, D), jnp.float32),
)

_tc2 = pl.pallas_call(
    _tc2_body,
    grid=(N // RB,),
    in_specs=[
        pl.BlockSpec((RB, D), lambda i: (i, 0)),
        pl.BlockSpec((RB, D), lambda i: (i, 0)),
        pl.BlockSpec((RB, 1), lambda i: (i, 0)),
        pl.BlockSpec((1, D), lambda i: (0, 0)),
        pl.BlockSpec((D, 2 * D), lambda i: (0, 0)),
    ],
    out_specs=pl.BlockSpec((RB, 2 * D), lambda i: (i, 0)),
    out_shape=jax.ShapeDtypeStruct((N, 2 * D), jnp.float32),
)

_tc3 = pl.pallas_call(
    _tc3_body,
    grid=(N // RB,),
    in_specs=[
        pl.BlockSpec((RB, 2 * D), lambda i: (i, 0)),
        pl.BlockSpec((RB, 2 * D), lambda i: (i, 0)),
        pl.BlockSpec((RB, 1), lambda i: (i, 0)),
        pl.BlockSpec((1, 2 * D), lambda i: (0, 0)),
    ],
    out_specs=pl.BlockSpec((1, 2 * D), lambda i: (0, 0)),
    out_shape=jax.ShapeDtypeStruct((1, 2 * D), jnp.float32),
)


def kernel(x, edge_index, W1, b1, W2, b2):
    src = edge_index[0].astype(jnp.int32)
    dst = edge_index[1].astype(jnp.int32)
    degp, src32, dst8 = _deg_kernel(src, dst)

    dinv = _tc0(degp)
    xws1 = _tc1(dinv, x, W1)
    agg1 = _agg32(xws1.reshape(N * 32, 8), src32, dst8)
    agg1t = agg1.reshape(32, N, 8).transpose(1, 0, 2).reshape(N, D)

    xws2 = _tc2(agg1t, xws1, dinv, b1.reshape(1, D), W2)
    agg2 = _agg64(xws2.reshape(N * 64, 8), src32 + src32, dst8)
    agg2t = agg2.reshape(64, N, 8).transpose(1, 0, 2).reshape(N, 2 * D)

    out = _tc3(agg2t, xws2, dinv, b2.reshape(1, 2 * D))
    return out.reshape(2 * D)


# speedup vs baseline: 8.0703x; 1.0499x over previous
import functools

import jax
import jax.numpy as jnp
from jax import lax
from jax.experimental import pallas as pl
from jax.experimental.pallas import tpu as pltpu
from jax.experimental.pallas import tpu_sc as plsc

N = 10000
D = 256
E = 160000
NW = 32
EPW = E // NW
B = 2000
RB = 1000

_mesh = lambda: plsc.VectorSubcoreMesh(
    core_axis_name="c", subcore_axis_name="s", num_cores=2, num_subcores=16)
_sc_params = pltpu.CompilerParams(
    needs_layout_passes=False, use_tc_tiling_on_sc=False)


@functools.partial(
    pl.kernel,
    out_type=(
        jax.ShapeDtypeStruct((NW, N), jnp.float32),
        jax.ShapeDtypeStruct((E,), jnp.int32),
        jax.ShapeDtypeStruct((E,), jnp.int32),
    ),
    mesh=_mesh(),
    scratch_types=[
        pltpu.VMEM((EPW + 16,), jnp.int32),
        pltpu.VMEM((EPW + 16,), jnp.int32),
        pltpu.VMEM((EPW + 16,), jnp.int32),
        pltpu.VMEM((8 * N,), jnp.float32),
    ],
    compiler_params=_sc_params,
)
def _deg_kernel(src_hbm, dst_hbm, degp_hbm, src32_hbm, dst8_hbm,
                srcb, dstb, d8b, acc):
    wid = lax.axis_index("s") * 2 + lax.axis_index("c")
    base = wid * EPW
    pltpu.sync_copy(src_hbm.at[pl.ds(base, EPW)], srcb.at[pl.ds(0, EPW)])
    pltpu.sync_copy(dst_hbm.at[pl.ds(base, EPW)], dstb.at[pl.ds(0, EPW)])

    lanes = lax.iota(jnp.int32, 16)
    offs = (lanes & 7) * N
    mlo = lanes < 8
    mhi = lanes >= 8
    ones = jnp.ones((16,), jnp.float32)
    zero = jnp.zeros((16,), jnp.float32)

    @plsc.parallel_loop(0, (8 * N) // 16, unroll=8)
    def _(i):
        acc[pl.ds(i * 16, 16)] = zero

    nfull = EPW // 16

    @plsc.parallel_loop(0, nfull, unroll=8)
    def _(i):
        j = i * 16
        sv = srcb[pl.ds(j, 16)]
        srcb[pl.ds(j, 16)] = sv * 32
        dv = dstb[pl.ds(j, 16)]
        d8b[pl.ds(j, 16)] = dv * 8
        addr = dv + offs
        plsc.addupdate_scatter(acc, [addr], ones, mask=mlo)
        plsc.addupdate_scatter(acc, [addr], ones, mask=mhi)

    j = nfull * 16
    sv = srcb[pl.ds(j, 16)]
    srcb[pl.ds(j, 16)] = sv * 32
    dv = dstb[pl.ds(j, 16)]
    d8b[pl.ds(j, 16)] = dv * 8
    plsc.addupdate_scatter(acc, [dv + offs], ones, mask=mlo)

    @plsc.parallel_loop(0, N // 16, unroll=4)
    def _(i):
        s = acc[pl.ds(i * 16, 16)]
        for k in range(1, 8):
            s = s + acc[pl.ds(k * N + i * 16, 16)]
        acc[pl.ds(i * 16, 16)] = s

    pltpu.sync_copy(srcb.at[pl.ds(0, EPW)], src32_hbm.at[pl.ds(base, EPW)])
    pltpu.sync_copy(d8b.at[pl.ds(0, EPW)], dst8_hbm.at[pl.ds(base, EPW)])
    pltpu.sync_copy(acc.at[pl.ds(0, N)], degp_hbm.at[wid])


def _make_agg(K):
    npass = K // NW

    nb = E // B
    nb2 = nb // 2

    @functools.partial(
        pl.kernel,
        out_type=jax.ShapeDtypeStruct((K, 8 * N), jnp.float32),
        mesh=_mesh(),
        scratch_types=[
            pltpu.VMEM((2, B), jnp.int32),
            pltpu.VMEM((2, B), jnp.int32),
            pltpu.VMEM((2, B, 8), jnp.float32),
            pltpu.VMEM((8 * N,), jnp.float32),
            pltpu.SemaphoreType.DMA,
            pltpu.SemaphoreType.DMA,
            pltpu.SemaphoreType.DMA,
            pltpu.SemaphoreType.DMA,
            pltpu.SemaphoreType.DMA,
            pltpu.SemaphoreType.DMA,
        ],
        compiler_params=_sc_params,
    )
    def agg(xflat_hbm, srck_hbm, dst8_hbm, out_hbm, idxb, dstb, rows, acc,
            s0, s1, g0, g1, d0, d1):
        wid = lax.axis_index("s") * 2 + lax.axis_index("c")
        lanes = lax.iota(jnp.int32, 16)
        feat = lanes & 7
        half = lanes >> 3
        mlo = lanes < 8
        mhi = lanes >= 8
        zero = jnp.zeros((16,), jnp.float32)

        def process(dref, rref):
            @plsc.parallel_loop(0, B // 2, unroll=8)
            def _(k):
                ev = half + 2 * k
                dstv = plsc.load_gather(dref, [ev])
                addr = dstv + feat
                row = plsc.load_gather(rref, [ev, feat])
                plsc.addupdate_scatter(acc, [addr], row)

        nview = N * K - K + 1
        for p in range(npass):
            chunk = wid + NW * p
            xview = xflat_hbm.at[pl.ds(chunk, nview)]

            @plsc.parallel_loop(0, (8 * N) // 16, unroll=8)
            def _(i):
                acc[pl.ds(i * 16, 16)] = zero

            pltpu.sync_copy(srck_hbm.at[pl.ds(0, B)], idxb.at[0])
            pltpu.async_copy(xview.at[idxb.at[0]], rows.at[0], g0)
            pltpu.async_copy(srck_hbm.at[pl.ds(B, B)], idxb.at[1], s1)
            pltpu.async_copy(dst8_hbm.at[pl.ds(0, B)], dstb.at[0], d0)
            pltpu.async_copy(dst8_hbm.at[pl.ds(B, B)], dstb.at[1], d1)

            def bb(k, _):
                off = 2 * k * B
                more = k < nb2 - 1
                pltpu.make_async_copy(
                    srck_hbm.at[pl.ds(off + B, B)], idxb.at[1], s1).wait()
                pltpu.async_copy(xview.at[idxb.at[1]], rows.at[1], g1)
                pltpu.make_async_copy(
                    xview.at[idxb.at[0]], rows.at[0], g0).wait()

                @pl.when(more)
                def _():
                    pltpu.async_copy(
                        srck_hbm.at[pl.ds(off + 2 * B, B)], idxb.at[0], s0)
                pltpu.make_async_copy(
                    dst8_hbm.at[pl.ds(off, B)], dstb.at[0], d0).wait()
                process(dstb.at[0], rows.at[0])

                @pl.when(more)
                def _():
                    pltpu.async_copy(
                        dst8_hbm.at[pl.ds(off + 2 * B, B)], dstb.at[0], d0)
                    pltpu.make_async_copy(
                        srck_hbm.at[pl.ds(off + 2 * B, B)], idxb.at[0], s0).wait()
                    pltpu.async_copy(xview.at[idxb.at[0]], rows.at[0], g0)
                pltpu.make_async_copy(
                    xview.at[idxb.at[1]], rows.at[1], g1).wait()

                @pl.when(more)
                def _():
                    pltpu.async_copy(
                        srck_hbm.at[pl.ds(off + 3 * B, B)], idxb.at[1], s1)
                pltpu.make_async_copy(
                    dst8_hbm.at[pl.ds(off + B, B)], dstb.at[1], d1).wait()
                process(dstb.at[1], rows.at[1])

                @pl.when(more)
                def _():
                    pltpu.async_copy(
                        dst8_hbm.at[pl.ds(off + 3 * B, B)], dstb.at[1], d1)
                return 0
            lax.fori_loop(0, nb2, bb, 0)

            pltpu.sync_copy(acc, out_hbm.at[chunk])
    return agg


_agg32 = _make_agg(32)
_agg64 = _make_agg(64)


def _tc0_body(degp_ref, dinv_ref):
    deg = jnp.sum(degp_ref[...], axis=0) + 1.0
    dinv_ref[...] = lax.rsqrt(deg)[:, None]


def _tc1_body(dinv_ref, x_ref, w1_ref, xws_ref):
    xw = jnp.dot(x_ref[...], w1_ref[...], preferred_element_type=jnp.float32)
    xws_ref[...] = xw * dinv_ref[...]


def _tc2_body(agg_ref, xws_ref, dinv_ref, b1_ref, w2_ref, xws2_ref):
    h = jnp.maximum(dinv_ref[...] * (agg_ref[...] + xws_ref[...]) + b1_ref[...], 0.0)
    xw2 = jnp.dot(h, w2_ref[...], preferred_element_type=jnp.float32)
    xws2_ref[...] = xw2 * dinv_ref[...]


def _tc3_body(agg2_ref, xws2_ref, dinv_ref, b2_ref, out_ref):
    i = pl.program_id(0)
    h2 = jnp.maximum(dinv_ref[...] * (agg2_ref[...] + xws2_ref[...]) + b2_ref[...], 0.0)
    part = jnp.sum(h2, axis=0, keepdims=True)

    @pl.when(i == 0)
    def _():
        out_ref[...] = part

    @pl.when(i > 0)
    def _():
        out_ref[...] = out_ref[...] + part

    @pl.when(i == N // RB - 1)
    def _():
        out_ref[...] = out_ref[...] * (1.0 / N)


_tc0 = pl.pallas_call(
    _tc0_body,
    in_specs=[pl.BlockSpec((NW, N), lambda: (0, 0))],
    out_specs=pl.BlockSpec((N, 1), lambda: (0, 0)),
    out_shape=jax.ShapeDtypeStruct((N, 1), jnp.float32),
)

_tc1 = pl.pallas_call(
    _tc1_body,
    grid=(N // RB,),
    in_specs=[
        pl.BlockSpec((RB, 1), lambda i: (i, 0)),
        pl.BlockSpec((RB, D), lambda i: (i, 0)),
        pl.BlockSpec((D, D), lambda i: (0, 0)),
    ],
    out_specs=pl.BlockSpec((RB, D), lambda i: (i, 0)),
    out_shape=jax.ShapeDtypeStruct((N, D), jnp.float32),
)

_tc2 = pl.pallas_call(
    _tc2_body,
    grid=(N // RB,),
    in_specs=[
        pl.BlockSpec((RB, D), lambda i: (i, 0)),
        pl.BlockSpec((RB, D), lambda i: (i, 0)),
        pl.BlockSpec((RB, 1), lambda i: (i, 0)),
        pl.BlockSpec((1, D), lambda i: (0, 0)),
        pl.BlockSpec((D, 2 * D), lambda i: (0, 0)),
    ],
    out_specs=pl.BlockSpec((RB, 2 * D), lambda i: (i, 0)),
    out_shape=jax.ShapeDtypeStruct((N, 2 * D), jnp.float32),
)

_tc3 = pl.pallas_call(
    _tc3_body,
    grid=(N // RB,),
    in_specs=[
        pl.BlockSpec((RB, 2 * D), lambda i: (i, 0)),
        pl.BlockSpec((RB, 2 * D), lambda i: (i, 0)),
        pl.BlockSpec((RB, 1), lambda i: (i, 0)),
        pl.BlockSpec((1, 2 * D), lambda i: (0, 0)),
    ],
    out_specs=pl.BlockSpec((1, 2 * D), lambda i: (0, 0)),
    out_shape=jax.ShapeDtypeStruct((1, 2 * D), jnp.float32),
)


def kernel(x, edge_index, W1, b1, W2, b2):
    src = edge_index[0].astype(jnp.int32)
    dst = edge_index[1].astype(jnp.int32)
    degp, src32, dst8 = _deg_kernel(src, dst)

    dinv = _tc0(degp)
    xws1 = _tc1(dinv, x, W1)
    agg1 = _agg32(xws1.reshape(N * 32, 8), src32, dst8)
    agg1t = agg1.reshape(32, N, 8).transpose(1, 0, 2).reshape(N, D)

    xws2 = _tc2(agg1t, xws1, dinv, b1.reshape(1, D), W2)
    agg2 = _agg64(xws2.reshape(N * 64, 8), src32 + src32, dst8)
    agg2t = agg2.reshape(64, N, 8).transpose(1, 0, 2).reshape(N, 2 * D)

    out = _tc3(agg2t, xws2, dinv, b2.reshape(1, 2 * D))
    return out.reshape(2 * D)
